# baseline (speedup 1.0000x reference)
import jax
import jax.numpy as jnp
from jax.experimental import pallas as pl


def _pos_kernel(row_ref, col_ref, out_ref, *, h, w, d):
    col = col_ref[:w, :]
    row = row_ref[:h, :]
    col_t = col.T
    row_t = row.T
    x_part = jnp.broadcast_to(col_t[:, None, :], (d, h, w))
    y_part = jnp.broadcast_to(row_t[:, :, None], (d, h, w))
    pos = jnp.concatenate([x_part, y_part], axis=0)
    out_ref[...] = pos[None]


def kernel(inputs, row_embed, col_embed):
    h, w = inputs.shape[-2], inputs.shape[-1]
    b = inputs.shape[0]
    d = row_embed.shape[1]

    out = pl.pallas_call(
        lambda r, c, o: _pos_kernel(r, c, o, h=h, w=w, d=d),
        grid=(b,),
        in_specs=[
            pl.BlockSpec(row_embed.shape, lambda i: (0, 0)),
            pl.BlockSpec(col_embed.shape, lambda i: (0, 0)),
        ],
        out_specs=pl.BlockSpec((1, 2 * d, h, w), lambda i: (i, 0, 0, 0)),
        out_shape=jax.ShapeDtypeStruct((b, 2 * d, h, w), jnp.float32),
    )(row_embed, col_embed)
    return out
